# Initial kernel scaffold; baseline (speedup 1.0000x reference)
import jax
import jax.numpy as jnp
from jax.experimental import pallas as pl


def kernel(x0, x1, x2, x3, c4conv_w, c4conv_shift, lat0_w, lat0_shift, lat1_w, lat1_shift, lat2_w, lat2_shift, fpn0_w, fpn0_shift, fpn1_w, fpn1_shift, fpn2_w, fpn2_shift, c5g0, c5g1, c5g2, c5g3, c5b, conv5_1_w, conv5_1_bias):
    raise NotImplementedError("write your pallas kernel here")


# speedup vs baseline: 1.3064x; 1.3064x over previous
import functools

import numpy as np
import jax
import jax.numpy as jnp
from jax.experimental import pallas as pl
from jax.experimental.pallas import tpu as pltpu

D = 256
TH = 8
NT = 128 // TH
_VMEM = 64 * 1024 * 1024


def _cp(n_axes):
    return pltpu.CompilerParams(
        dimension_semantics=("parallel",) * n_axes,
        vmem_limit_bytes=_VMEM)


@functools.lru_cache(maxsize=None)
def _bilin(out_size, in_size, pad=0):
    if out_size == 1:
        src = np.zeros((1,), np.float64)
    else:
        src = np.arange(out_size, dtype=np.float64) * (in_size - 1) / (out_size - 1)
    i0 = np.clip(np.floor(src).astype(np.int64), 0, in_size - 1)
    i1 = np.clip(i0 + 1, 0, in_size - 1)
    frac = src - i0
    m = np.zeros((out_size, in_size), np.float64)
    rows = np.arange(out_size)
    m[rows, i0] += 1.0 - frac
    m[rows, i1] += frac
    if pad:
        m = np.concatenate([np.zeros((pad, in_size)), m,
                            np.zeros((pad, in_size))], axis=0)
    return m.astype(np.float32)


def _c4_body(x_ref, w_ref, b_ref, o_ref):
    acc = None
    for ky in range(3):
        for kx in range(3):
            xs = x_ref[0, ky:ky + 16, kx:kx + 16, :].reshape(256, 2048)
            p = jnp.dot(xs, w_ref[ky * 3 + kx],
                        preferred_element_type=jnp.float32)
            acc = p if acc is None else acc + p
    acc = jnp.maximum(acc + b_ref[...], 0.0)
    o_ref[0] = acc.reshape(16, 16, D).astype(jnp.bfloat16)


def _hpass_body(x_ref, ah_ref, o_ref):
    o_ref[0] = jnp.dot(ah_ref[...], x_ref[0].astype(jnp.float32),
                       preferred_element_type=jnp.float32)


def _conv3x3_val(xpad, w9, rows):
    wd = xpad.shape[1] - 2
    cin = xpad.shape[2]
    acc = None
    for ky in range(3):
        for kx in range(3):
            xs = xpad[ky:ky + rows, kx:kx + wd, :].reshape(rows * wd, cin)
            p = jnp.dot(xs, w9[ky * 3 + kx],
                        preferred_element_type=jnp.float32)
            acc = p if acc is None else acc + p
    return acc


def _level_body(t_ref, x_ref, latw_ref, latb_ref, aw_ref, fw_ref, fb_ref,
                feat_ref, out_ref, *, h, w_src, cin):
    lat = jax.lax.dot_general(
        x_ref[0].astype(jnp.bfloat16), latw_ref[...],
        dimension_numbers=(((0,), (0,)), ((), ())),
        preferred_element_type=jnp.float32)
    lat = jnp.maximum(lat + latb_ref[...], 0.0).astype(jnp.bfloat16)
    lat = lat.reshape(h, h, D)
    aw = aw_ref[...]
    awb = jnp.broadcast_to(aw[None], (h,) + aw.shape)
    up = jnp.einsum('row,rwc->roc', awb, t_ref[0],
                    preferred_element_type=jnp.float32)
    feat = (up + lat.astype(jnp.float32)).astype(jnp.bfloat16)
    feat_ref[0] = feat
    xpad = jnp.pad(feat, ((1, 1), (1, 1), (0, 0)))
    acc = _conv3x3_val(xpad, fw_ref, h) + fb_ref[...]
    out_ref[0] = jnp.maximum(acc, 0.0).reshape(h, h, D).astype(jnp.bfloat16)


def _lat0_body(x_ref, w_ref, b_ref, o_ref):
    acc = jax.lax.dot_general(
        x_ref[0].astype(jnp.bfloat16), w_ref[...],
        dimension_numbers=(((0,), (0,)), ((), ())),
        preferred_element_type=jnp.float32)
    acc = jnp.maximum(acc + b_ref[...], 0.0)
    o_ref[0] = acc.astype(jnp.bfloat16)


def _row_mask(start, n, lo, hi):
    r = jax.lax.broadcasted_iota(jnp.int32, (n, 1, 1), 0) + start
    return jnp.where((r >= lo) & (r < hi), 1.0, 0.0).astype(jnp.float32)


def _k4_body(t0_ref, lat0_ref, t16_ref, t32_ref, t64_ref,
             aw16_ref, aw32_ref, aw64_ref,
             f0w_ref, f0b_ref,
             g0_ref, g1_ref, g2_ref, g3_ref, c5b_ref,
             cw_ref, cb_ref, y_ref, acc_ref):
    t = pl.program_id(1)
    base = t * TH

    def wpass(aw_ref_, src, rows):
        aw = aw_ref_[...]
        awb = jnp.broadcast_to(aw[None], (rows,) + aw.shape)
        return jnp.einsum('row,rwc->roc', awb, src,
                          preferred_element_type=jnp.float32)

    def add_group(g, w9_ref, first):
        gp = jnp.pad(g, ((0, 0), (1, 1), (0, 0)))
        p = _conv3x3_val(gp, w9_ref, TH)
        acc_ref[...] = p if first else acc_ref[...] + p

    add_group(wpass(aw16_ref, t16_ref[0], TH + 2).astype(jnp.bfloat16),
              g0_ref, True)
    add_group(wpass(aw32_ref, t32_ref[0], TH + 2).astype(jnp.bfloat16),
              g1_ref, False)
    add_group(wpass(aw64_ref, t64_ref[0], TH + 2).astype(jnp.bfloat16),
              g2_ref, False)
    up0 = wpass(aw64_ref, t0_ref[0], TH + 4)
    feat = (up0 + lat0_ref[0].astype(jnp.float32)).astype(jnp.bfloat16)
    fpad = jnp.pad(feat, ((0, 0), (1, 1), (0, 0)))
    acc0 = _conv3x3_val(fpad, f0w_ref, TH + 2) + f0b_ref[...]
    out0 = jnp.maximum(acc0, 0.0).reshape(TH + 2, 128, D)
    out0 = (out0 * _row_mask(base - 1, TH + 2, 0, 128)).astype(jnp.bfloat16)
    add_group(out0, g3_ref, False)
    h5 = jnp.maximum(acc_ref[...] + c5b_ref[...], 0.0).astype(jnp.bfloat16)
    y = jax.lax.dot_general(
        cw_ref[...], h5,
        dimension_numbers=(((0,), (1,)), ((), ())),
        preferred_element_type=jnp.float32)
    y_ref[0] = y + cb_ref[...]


def _c4conv(x3, w9, shift):
    n = x3.shape[0]
    x = jnp.transpose(x3, (0, 2, 3, 1)).astype(jnp.bfloat16)
    x = jnp.pad(x, ((0, 0), (1, 1), (1, 1), (0, 0)))
    return pl.pallas_call(
        _c4_body,
        out_shape=jax.ShapeDtypeStruct((n, 16, 16, D), jnp.bfloat16),
        grid=(n,),
        in_specs=[pl.BlockSpec((1, 18, 18, 2048), lambda b: (b, 0, 0, 0)),
                  pl.BlockSpec(w9.shape, lambda b: (0, 0, 0)),
                  pl.BlockSpec((1, D), lambda b: (0, 0))],
        out_specs=pl.BlockSpec((1, 16, 16, D), lambda b: (b, 0, 0, 0)),
        compiler_params=_cp(1),
    )(x, w9, shift.reshape(1, D))


def _hpass(x_flat, ho, pad=0):
    n, h, L = x_flat.shape
    hp = ho + 2 * pad
    ah = jnp.asarray(_bilin(ho, h, pad))
    return pl.pallas_call(
        _hpass_body,
        out_shape=jax.ShapeDtypeStruct((n, hp, L), jnp.float32),
        grid=(n,),
        in_specs=[pl.BlockSpec((1, h, L), lambda b: (b, 0, 0)),
                  pl.BlockSpec((hp, h), lambda b: (0, 0))],
        out_specs=pl.BlockSpec((1, hp, L), lambda b: (b, 0, 0)),
        compiler_params=_cp(1),
    )(x_flat, ah)


def _level(t, x, latw, latb, fw, fb, h, cin):
    n = t.shape[0]
    w_src = h // 2
    aw = jnp.asarray(_bilin(h, w_src))
    t4 = t.reshape(n, h, w_src, D)
    feat, out = pl.pallas_call(
        functools.partial(_level_body, h=h, w_src=w_src, cin=cin),
        out_shape=(jax.ShapeDtypeStruct((n, h, h, D), jnp.bfloat16),
                   jax.ShapeDtypeStruct((n, h, h, D), jnp.bfloat16)),
        grid=(n,),
        in_specs=[pl.BlockSpec((1, h, w_src, D), lambda b: (b, 0, 0, 0)),
                  pl.BlockSpec((1, cin, h * h), lambda b: (b, 0, 0)),
                  pl.BlockSpec((cin, D), lambda b: (0, 0)),
                  pl.BlockSpec((1, D), lambda b: (0, 0)),
                  pl.BlockSpec((h, w_src), lambda b: (0, 0)),
                  pl.BlockSpec(fw.shape, lambda b: (0, 0, 0)),
                  pl.BlockSpec((1, D), lambda b: (0, 0))],
        out_specs=(pl.BlockSpec((1, h, h, D), lambda b: (b, 0, 0, 0)),
                   pl.BlockSpec((1, h, h, D), lambda b: (b, 0, 0, 0))),
        compiler_params=_cp(1),
    )(t4, x.reshape(n, cin, h * h), latw, latb.reshape(1, D), aw, fw,
      fb.reshape(1, D))
    return feat, out


def _lat0(x0, w, shift):
    n = x0.shape[0]
    hw = 128 * 128
    tm = hw // 4
    out = pl.pallas_call(
        _lat0_body,
        out_shape=jax.ShapeDtypeStruct((n, hw, D), jnp.bfloat16),
        grid=(n, 4),
        in_specs=[pl.BlockSpec((1, 256, tm), lambda b, t: (b, 0, t)),
                  pl.BlockSpec((256, D), lambda b, t: (0, 0)),
                  pl.BlockSpec((1, D), lambda b, t: (0, 0))],
        out_specs=pl.BlockSpec((1, tm, D), lambda b, t: (b, t, 0)),
        compiler_params=_cp(2),
    )(x0.reshape(n, 256, hw), w, shift.reshape(1, D))
    return out.reshape(n, 128, 128, D)


def _k4(t0, lat0, t16, t32, t64, f0w, f0b, g0, g1, g2, g3, c5b, cw, cb):
    n = t0.shape[0]
    aw16 = jnp.asarray(_bilin(128, 16))
    aw32 = jnp.asarray(_bilin(128, 32))
    aw64 = jnp.asarray(_bilin(128, 64))

    def espec(rows, ofs, w):
        return pl.BlockSpec(
            (pl.Element(1), pl.Element(rows), pl.Element(w), pl.Element(D)),
            lambda b, t, _o=ofs: (b, t * TH + _o, 0, 0))

    y = pl.pallas_call(
        _k4_body,
        out_shape=jax.ShapeDtypeStruct((n, 150, 128 * 128), jnp.float32),
        grid=(n, NT),
        in_specs=[
            espec(TH + 4, 0, 64),
            espec(TH + 4, 0, 128),
            espec(TH + 2, 1, 16),
            espec(TH + 2, 1, 32),
            espec(TH + 2, 1, 64),
            pl.BlockSpec((128, 16), lambda b, t: (0, 0)),
            pl.BlockSpec((128, 32), lambda b, t: (0, 0)),
            pl.BlockSpec((128, 64), lambda b, t: (0, 0)),
            pl.BlockSpec(f0w.shape, lambda b, t: (0, 0, 0)),
            pl.BlockSpec((1, D), lambda b, t: (0, 0)),
            pl.BlockSpec(g0.shape, lambda b, t: (0, 0, 0)),
            pl.BlockSpec(g1.shape, lambda b, t: (0, 0, 0)),
            pl.BlockSpec(g2.shape, lambda b, t: (0, 0, 0)),
            pl.BlockSpec(g3.shape, lambda b, t: (0, 0, 0)),
            pl.BlockSpec((1, 512), lambda b, t: (0, 0)),
            pl.BlockSpec(cw.shape, lambda b, t: (0, 0)),
            pl.BlockSpec((150, 1), lambda b, t: (0, 0)),
        ],
        out_specs=pl.BlockSpec((1, 150, TH * 128), lambda b, t: (b, 0, t)),
        scratch_shapes=[pltpu.VMEM((TH * 128, 512), jnp.float32)],
        compiler_params=_cp(2),
    )(t0.reshape(n, 132, 64, D), lat0, t16.reshape(n, 132, 16, D),
      t32.reshape(n, 132, 32, D), t64.reshape(n, 132, 64, D),
      aw16, aw32, aw64, f0w, f0b.reshape(1, D),
      g0, g1, g2, g3, c5b.reshape(1, 512), cw, cb.reshape(150, 1))
    return y.reshape(n, 150, 128, 128)


def kernel(x0, x1, x2, x3, c4conv_w, c4conv_shift, lat0_w, lat0_shift,
           lat1_w, lat1_shift, lat2_w, lat2_shift, fpn0_w, fpn0_shift,
           fpn1_w, fpn1_shift, fpn2_w, fpn2_shift, c5g0, c5g1, c5g2, c5g3,
           c5b, conv5_1_w, conv5_1_bias):
    n = x0.shape[0]
    feat16 = _c4conv(x3, c4conv_w, c4conv_shift)
    t2 = _hpass(feat16.reshape(n, 16, 16 * D), 32)
    feat32, out2 = _level(t2, x2, lat2_w, lat2_shift, fpn2_w, fpn2_shift,
                          32, 1024)
    t1 = _hpass(feat32.reshape(n, 32, 32 * D), 64)
    feat64, out1 = _level(t1, x1, lat1_w, lat1_shift, fpn1_w, fpn1_shift,
                          64, 512)
    lat0 = _lat0(x0, lat0_w, lat0_shift)
    lat0 = jnp.pad(lat0, ((0, 0), (2, 2), (0, 0), (0, 0)))
    t0 = _hpass(feat64.reshape(n, 64, 64 * D), 128, pad=2)
    t16 = _hpass(feat16.reshape(n, 16, 16 * D), 128, pad=2)
    t32 = _hpass(out2.reshape(n, 32, 32 * D), 128, pad=2)
    t64 = _hpass(out1.reshape(n, 64, 64 * D), 128, pad=2)
    y = _k4(t0, lat0, t16, t32, t64, fpn0_w, fpn0_shift,
            c5g0, c5g1, c5g2, c5g3, c5b, conv5_1_w, conv5_1_bias)
    return (y,)
